# Initial kernel scaffold; baseline (speedup 1.0000x reference)
import jax
import jax.numpy as jnp
from jax.experimental import pallas as pl


def kernel(vals, idxs, kept_idx, keepRate):
    raise NotImplementedError("write your pallas kernel here")


# speedup vs baseline: 18.1237x; 18.1237x over previous
import functools

import jax
import jax.numpy as jnp
from jax import lax
from jax.experimental import pallas as pl
from jax.experimental.pallas import tpu as pltpu
from jax.experimental.pallas import tpu_sc as plsc

_NC = 2
_NS = 16
_NW = _NC * _NS
_LANES = 16
_CHUNK = 8192
_UNROLL = 8


def _vec_loop(n_vec, body_u):
    n_outer = n_vec // _UNROLL
    if n_outer > 0:
        def outer(i, carry):
            b = i * (_LANES * _UNROLL)
            for u in range(_UNROLL):
                body_u(b + u * _LANES)
            return carry
        lax.fori_loop(0, n_outer, outer, 0, unroll=False)
    for u in range(n_vec % _UNROLL):
        body_u(n_outer * _LANES * _UNROLL + u * _LANES)


@functools.lru_cache(maxsize=None)
def _build(E, K):
    C = _CHUNK
    nc_full = K // C
    rem = K - nc_full * C
    n_vec_c = C // _LANES
    n_vec_t = (rem + _LANES - 1) // _LANES
    mesh = plsc.VectorSubcoreMesh(core_axis_name="c", subcore_axis_name="s")

    @functools.partial(
        pl.kernel,
        out_type=[
            jax.ShapeDtypeStruct((2 * K,), jnp.int32),
            jax.ShapeDtypeStruct((K,), jnp.float32),
        ],
        mesh=mesh,
        scratch_types=[
            pltpu.VMEM((C,), jnp.int32),
            pltpu.VMEM((C,), jnp.int32),
            pltpu.VMEM((C,), jnp.int32),
            pltpu.VMEM((C,), jnp.float32),
            pltpu.VMEM((C,), jnp.int32),
            pltpu.VMEM((C,), jnp.int32),
            pltpu.VMEM((_LANES,), jnp.float32),
            pltpu.SemaphoreType.DMA,
            pltpu.SemaphoreType.DMA,
            pltpu.SemaphoreType.DMA,
        ],
    )
    def sc_kernel(vals_hbm, idxsf_hbm, kept_hbm, inv_hbm,
                  out_idxs_hbm, out_vals_hbm,
                  idx_v, idx2_v, pos_v, val_v, i0_v, i1_v, inv_v,
                  sem0, sem1, sem2):
        w = lax.axis_index("s") * _NC + lax.axis_index("c")
        pltpu.sync_copy(inv_hbm, inv_v)
        inv = inv_v[...]
        iota16 = lax.iota(jnp.int32, 16)

        def do_chunk(base, n, n_vec):
            pltpu.sync_copy(kept_hbm.at[pl.ds(base, n)], idx_v.at[pl.ds(0, n)])

            def prep(s):
                idx2_v[pl.ds(s, _LANES)] = idx_v[pl.ds(s, _LANES)] + E
                pos_v[pl.ds(s, _LANES)] = iota16 + (K + base + s)
            _vec_loop(n_vec, prep)

            cp0 = pltpu.async_copy(vals_hbm.at[idx_v.at[pl.ds(0, n)]],
                                   val_v.at[pl.ds(0, n)], sem0)
            cp1 = pltpu.async_copy(idxsf_hbm.at[idx_v.at[pl.ds(0, n)]],
                                   i0_v.at[pl.ds(0, n)], sem1)
            cp2 = pltpu.async_copy(idxsf_hbm.at[idx2_v.at[pl.ds(0, n)]],
                                   i1_v.at[pl.ds(0, n)], sem2)
            cp0.wait()
            cp1.wait()
            cp2.wait()

            def scale(s):
                val_v[pl.ds(s, _LANES)] = val_v[pl.ds(s, _LANES)] * inv
            _vec_loop(n_vec, scale)

            pltpu.sync_copy(val_v.at[pl.ds(0, n)],
                            out_vals_hbm.at[pl.ds(base, n)])
            pltpu.sync_copy(i0_v.at[pl.ds(0, n)],
                            out_idxs_hbm.at[pl.ds(base, n)])
            cp3 = pltpu.async_copy(i1_v.at[pl.ds(0, n)],
                                   out_idxs_hbm.at[pos_v.at[pl.ds(0, n)]],
                                   sem2)
            cp3.wait()

        n_base = nc_full // _NW
        n_extra = nc_full % _NW
        n_w = n_base + jnp.where(w < n_extra, 1, 0)

        def loop_body(i, carry):
            do_chunk((w + i * _NW) * C, C, n_vec_c)
            return carry
        lax.fori_loop(0, n_w, loop_body, 0, unroll=False)

        if rem > 0:
            @pl.when(w == _NW - 1)
            def _tail():
                do_chunk(nc_full * C, rem, n_vec_t)

    return sc_kernel


def kernel(vals, idxs, kept_idx, keepRate):
    E = vals.shape[0]
    K = kept_idx.shape[0]
    inv = jnp.full((_LANES,), 1.0, dtype=jnp.float32) / jnp.asarray(
        keepRate, dtype=jnp.float32)
    idxs_flat = idxs.reshape(-1)
    out_idxs_flat, out_vals = _build(E, K)(vals, idxs_flat, kept_idx, inv)
    return (out_idxs_flat.reshape(2, K), out_vals)


# speedup vs baseline: 220.0066x; 12.1392x over previous
import functools

import jax
import jax.numpy as jnp
from jax import lax
from jax.experimental import pallas as pl
from jax.experimental.pallas import tpu as pltpu
from jax.experimental.pallas import tpu_sc as plsc

_NC = 2
_NS = 16
_NW = _NC * _NS
_LANES = 16
_CHUNK = 8192
_UNROLL = 8


def _vec_loop(n_vec, body_u):
    n_outer = n_vec // _UNROLL
    if n_outer > 0:
        def outer(i, carry):
            b = i * (_LANES * _UNROLL)
            for u in range(_UNROLL):
                body_u(b + u * _LANES)
            return carry
        lax.fori_loop(0, n_outer, outer, 0, unroll=False)
    for u in range(n_vec % _UNROLL):
        body_u(n_outer * _LANES * _UNROLL + u * _LANES)


@functools.lru_cache(maxsize=None)
def _build(E, K):
    C = _CHUNK
    nc_full = K // C
    rem = K - nc_full * C
    n_vec_c = C // _LANES
    n_vec_t = (rem + _LANES - 1) // _LANES
    mesh = plsc.VectorSubcoreMesh(core_axis_name="c", subcore_axis_name="s")

    @functools.partial(
        pl.kernel,
        out_type=[
            jax.ShapeDtypeStruct((K,), jnp.int32),
            jax.ShapeDtypeStruct((K,), jnp.int32),
            jax.ShapeDtypeStruct((K,), jnp.float32),
        ],
        mesh=mesh,
        scratch_types=[
            pltpu.VMEM((C,), jnp.int32),
            pltpu.VMEM((C,), jnp.float32),
            pltpu.VMEM((C,), jnp.int32),
            pltpu.VMEM((C,), jnp.int32),
            pltpu.VMEM((_LANES,), jnp.float32),
            pltpu.SemaphoreType.DMA,
            pltpu.SemaphoreType.DMA,
            pltpu.SemaphoreType.DMA,
        ],
    )
    def sc_kernel(vals_hbm, row0_hbm, row1_hbm, kept_hbm, inv_hbm,
                  out_r0_hbm, out_r1_hbm, out_vals_hbm,
                  idx_v, val_v, i0_v, i1_v, inv_v,
                  sem0, sem1, sem2):
        w = lax.axis_index("s") * _NC + lax.axis_index("c")
        pltpu.sync_copy(inv_hbm, inv_v)
        inv = inv_v[...]

        def do_chunk(base, n, n_vec):
            pltpu.sync_copy(kept_hbm.at[pl.ds(base, n)], idx_v.at[pl.ds(0, n)])

            cp0 = pltpu.async_copy(vals_hbm.at[idx_v.at[pl.ds(0, n)]],
                                   val_v.at[pl.ds(0, n)], sem0)
            cp1 = pltpu.async_copy(row0_hbm.at[idx_v.at[pl.ds(0, n)]],
                                   i0_v.at[pl.ds(0, n)], sem1)
            cp2 = pltpu.async_copy(row1_hbm.at[idx_v.at[pl.ds(0, n)]],
                                   i1_v.at[pl.ds(0, n)], sem2)
            cp0.wait()
            cp1.wait()
            cp2.wait()

            def scale(s):
                val_v[pl.ds(s, _LANES)] = val_v[pl.ds(s, _LANES)] * inv
            _vec_loop(n_vec, scale)

            pltpu.sync_copy(val_v.at[pl.ds(0, n)],
                            out_vals_hbm.at[pl.ds(base, n)])
            pltpu.sync_copy(i0_v.at[pl.ds(0, n)],
                            out_r0_hbm.at[pl.ds(base, n)])
            pltpu.sync_copy(i1_v.at[pl.ds(0, n)],
                            out_r1_hbm.at[pl.ds(base, n)])

        n_base = nc_full // _NW
        n_extra = nc_full % _NW
        n_w = n_base + jnp.where(w < n_extra, 1, 0)

        def loop_body(i, carry):
            do_chunk((w + i * _NW) * C, C, n_vec_c)
            return carry
        lax.fori_loop(0, n_w, loop_body, 0, unroll=False)

        if rem > 0:
            @pl.when(w == _NW - 1)
            def _tail():
                do_chunk(nc_full * C, rem, n_vec_t)

    return sc_kernel


def kernel(vals, idxs, kept_idx, keepRate):
    E = vals.shape[0]
    K = kept_idx.shape[0]
    inv = jnp.full((_LANES,), 1.0, dtype=jnp.float32) / jnp.asarray(
        keepRate, dtype=jnp.float32)
    out_r0, out_r1, out_vals = _build(E, K)(
        vals, idxs[0], idxs[1], kept_idx, inv)
    return (jnp.stack([out_r0, out_r1]), out_vals)


# speedup vs baseline: 221.3105x; 1.0059x over previous
import functools

import jax
import jax.numpy as jnp
from jax import lax
from jax.experimental import pallas as pl
from jax.experimental.pallas import tpu as pltpu
from jax.experimental.pallas import tpu_sc as plsc

_NC = 2
_NS = 16
_NW = _NC * _NS
_LANES = 16
_CHUNK = 16384
_UNROLL = 8


def _vec_loop(n_vec, body_u):
    n_outer = n_vec // _UNROLL
    if n_outer > 0:
        def outer(i, carry):
            b = i * (_LANES * _UNROLL)
            for u in range(_UNROLL):
                body_u(b + u * _LANES)
            return carry
        lax.fori_loop(0, n_outer, outer, 0, unroll=False)
    for u in range(n_vec % _UNROLL):
        body_u(n_outer * _LANES * _UNROLL + u * _LANES)


@functools.lru_cache(maxsize=None)
def _build(E, K):
    C = _CHUNK
    nc_full = K // C
    rem = K - nc_full * C
    n_vec_c = C // _LANES
    n_vec_t = (rem + _LANES - 1) // _LANES
    mesh = plsc.VectorSubcoreMesh(core_axis_name="c", subcore_axis_name="s")

    @functools.partial(
        pl.kernel,
        out_type=[
            jax.ShapeDtypeStruct((K,), jnp.int32),
            jax.ShapeDtypeStruct((K,), jnp.int32),
            jax.ShapeDtypeStruct((K,), jnp.float32),
        ],
        mesh=mesh,
        scratch_types=[
            pltpu.VMEM((C,), jnp.int32),
            pltpu.VMEM((C,), jnp.float32),
            pltpu.VMEM((C,), jnp.int32),
            pltpu.VMEM((C,), jnp.int32),
            pltpu.VMEM((_LANES,), jnp.float32),
            pltpu.SemaphoreType.DMA,
            pltpu.SemaphoreType.DMA,
            pltpu.SemaphoreType.DMA,
        ],
    )
    def sc_kernel(vals_hbm, row0_hbm, row1_hbm, kept_hbm, inv_hbm,
                  out_r0_hbm, out_r1_hbm, out_vals_hbm,
                  idx_v, val_v, i0_v, i1_v, inv_v,
                  sem0, sem1, sem2):
        w = lax.axis_index("s") * _NC + lax.axis_index("c")
        pltpu.sync_copy(inv_hbm, inv_v)
        inv = inv_v[...]

        def do_chunk(base, n, n_vec):
            pltpu.sync_copy(kept_hbm.at[pl.ds(base, n)], idx_v.at[pl.ds(0, n)])

            cp0 = pltpu.async_copy(vals_hbm.at[idx_v.at[pl.ds(0, n)]],
                                   val_v.at[pl.ds(0, n)], sem0)
            cp1 = pltpu.async_copy(row0_hbm.at[idx_v.at[pl.ds(0, n)]],
                                   i0_v.at[pl.ds(0, n)], sem1)
            cp2 = pltpu.async_copy(row1_hbm.at[idx_v.at[pl.ds(0, n)]],
                                   i1_v.at[pl.ds(0, n)], sem2)
            cp0.wait()
            cp1.wait()
            cp2.wait()

            def scale(s):
                val_v[pl.ds(s, _LANES)] = val_v[pl.ds(s, _LANES)] * inv
            _vec_loop(n_vec, scale)

            pltpu.sync_copy(val_v.at[pl.ds(0, n)],
                            out_vals_hbm.at[pl.ds(base, n)])
            pltpu.sync_copy(i0_v.at[pl.ds(0, n)],
                            out_r0_hbm.at[pl.ds(base, n)])
            pltpu.sync_copy(i1_v.at[pl.ds(0, n)],
                            out_r1_hbm.at[pl.ds(base, n)])

        n_base = nc_full // _NW
        n_extra = nc_full % _NW
        n_w = n_base + jnp.where(w < n_extra, 1, 0)

        def loop_body(i, carry):
            do_chunk((w + i * _NW) * C, C, n_vec_c)
            return carry
        lax.fori_loop(0, n_w, loop_body, 0, unroll=False)

        if rem > 0:
            @pl.when(w == _NW - 1)
            def _tail():
                do_chunk(nc_full * C, rem, n_vec_t)

    return sc_kernel


def kernel(vals, idxs, kept_idx, keepRate):
    E = vals.shape[0]
    K = kept_idx.shape[0]
    inv = jnp.full((_LANES,), 1.0, dtype=jnp.float32) / jnp.asarray(
        keepRate, dtype=jnp.float32)
    out_r0, out_r1, out_vals = _build(E, K)(
        vals, idxs[0], idxs[1], kept_idx, inv)
    return (jnp.stack([out_r0, out_r1]), out_vals)
